# baseline (speedup 1.0000x reference)
import functools

import jax
import jax.numpy as jnp
from jax import lax
from jax.experimental import pallas as pl
from jax.experimental.pallas import tpu as pltpu
from jax.experimental.pallas import tpu_sc as plsc

DIM = 16
BATCH = 16384
NUM_CORES = 2
NUM_SUBCORES = 16
NW = NUM_CORES * NUM_SUBCORES
BPW = BATCH // NW
NBLK = BPW // 16

_mesh = plsc.VectorSubcoreMesh(
    core_axis_name="c", subcore_axis_name="s",
    num_cores=NUM_CORES, num_subcores=NUM_SUBCORES)


@functools.partial(
    pl.kernel,
    out_type=jax.ShapeDtypeStruct((BATCH,), jnp.float32),
    mesh=_mesh,
    scratch_types=[
        pltpu.VMEM((BPW,), jnp.int32),
        pltpu.VMEM((BPW,), jnp.int32),
        pltpu.VMEM((BPW, DIM), jnp.float32),
        pltpu.VMEM((BPW, DIM), jnp.float32),
        pltpu.VMEM((BPW,), jnp.float32),
        pltpu.SemaphoreType.DMA,
        pltpu.SemaphoreType.DMA,
    ],
    compiler_params=pltpu.CompilerParams(
        needs_layout_passes=False, use_tc_tiling_on_sc=False),
)
def _cosine_sc(e_id_hbm, o_id_hbm, mentees_hbm, mentors_hbm, out_hbm,
               eidx_v, oidx_v, erows_v, orows_v, out_v, sem_e, sem_o):
    wid = lax.axis_index("s") * NUM_CORES + lax.axis_index("c")
    base = wid * BPW

    pltpu.sync_copy(e_id_hbm.at[pl.ds(base, BPW)], eidx_v)
    pltpu.sync_copy(o_id_hbm.at[pl.ds(base, BPW)], oidx_v)
    cp_e = pltpu.async_copy(mentees_hbm.at[eidx_v], erows_v, sem_e)
    cp_o = pltpu.async_copy(mentors_hbm.at[oidx_v], orows_v, sem_o)
    cp_e.wait()
    cp_o.wait()

    lanes = lax.iota(jnp.int32, 16)

    def block_body(b, carry):
        rows = b * 16 + lanes
        dot = jnp.zeros((16,), jnp.float32)
        ee = jnp.zeros((16,), jnp.float32)
        oo = jnp.zeros((16,), jnp.float32)
        for d in range(DIM):
            col = jnp.full((16,), d, jnp.int32)
            ge = plsc.load_gather(erows_v, [rows, col])
            go = plsc.load_gather(orows_v, [rows, col])
            dot = dot + ge * go
            ee = ee + ge * ge
            oo = oo + go * go
        x = ee * oo
        i = plsc.bitcast(x, jnp.int32)
        i = jnp.int32(0x5F3759DF) - lax.shift_right_logical(i, 1)
        y = plsc.bitcast(i, jnp.float32)
        hx = x * jnp.float32(0.5)
        for _ in range(3):
            y = y * (jnp.float32(1.5) - hx * y * y)
        out_v[pl.ds(b * 16, 16)] = dot * y
        return carry

    lax.fori_loop(0, NBLK, block_body, jnp.int32(0))

    pltpu.sync_copy(out_v, out_hbm.at[pl.ds(base, BPW)])


def kernel(e_id, o_id, mentees, mentors):
    return _cosine_sc(e_id, o_id, mentees, mentors)
